# baseline (speedup 1.0000x reference)
import jax
import jax.numpy as jnp
from jax.experimental import pallas as pl
from jax.experimental.pallas import tpu as pltpu

_EMB = 2048
_NE = 64
_TOK = 16384
_COEFF = 0.001
_BT = 512


def _router_kernel(x_ref, w_ref, b_ref, probs_ref, z_ref, zacc_ref):
    i = pl.program_id(0)
    nblk = _TOK // _BT
    logits = jax.lax.dot_general(
        x_ref[...], w_ref[...],
        dimension_numbers=(((1,), (1,)), ((), ())),
        preferred_element_type=jnp.float32,
    ) + b_ref[...]
    m = jnp.max(logits, axis=-1, keepdims=True)
    e = jnp.exp(logits - m)
    s = jnp.sum(e, axis=-1, keepdims=True)
    probs_ref[...] = e / s
    part = jnp.sum(logits * logits).reshape(1, 1)

    @pl.when(i == 0)
    def _init():
        zacc_ref[...] = part

    @pl.when(i > 0)
    def _acc():
        zacc_ref[...] += part

    @pl.when(i == nblk - 1)
    def _fin():
        z_ref[...] = zacc_ref[...] * (_COEFF / (_TOK * _NE))


def kernel(x, W, b):
    nblk = _TOK // _BT
    probs, z = pl.pallas_call(
        _router_kernel,
        grid=(nblk,),
        in_specs=[
            pl.BlockSpec((_BT, _EMB), lambda i: (i, 0)),
            pl.BlockSpec((_NE, _EMB), lambda i: (0, 0)),
            pl.BlockSpec((1, _NE), lambda i: (0, 0)),
        ],
        out_specs=[
            pl.BlockSpec((_BT, _NE), lambda i: (i, 0)),
            pl.BlockSpec((1, 1), lambda i: (0, 0)),
        ],
        out_shape=[
            jax.ShapeDtypeStruct((_TOK, _NE), jnp.float32),
            jax.ShapeDtypeStruct((1, 1), jnp.float32),
        ],
        scratch_shapes=[
            pltpu.VMEM((1, 1), jnp.float32),
        ],
        compiler_params=pltpu.CompilerParams(
            dimension_semantics=("arbitrary",),
        ),
    )(x, W, b.reshape(1, _NE))
    return (probs, z.reshape(()))


# speedup vs baseline: 1.0958x; 1.0958x over previous
import jax
import jax.numpy as jnp
from jax.experimental import pallas as pl
from jax.experimental.pallas import tpu as pltpu

_EMB = 2048
_NE = 64
_TOK = 16384
_COEFF = 0.001
_BT = 1024
_HALF = _TOK // 2
_K = 2


def _copy_in(x_hbm, xbuf, sems, blk, slot):
    pltpu.make_async_copy(
        x_hbm.at[pl.ds(_HALF + blk * _BT, _BT), :],
        xbuf.at[slot],
        sems.at[slot],
    ).start()


def _router_kernel(xa_ref, x_hbm, w_ref, b_ref, probs_ref, z_ref,
                   xbuf, sems, zacc_ref):
    i = pl.program_id(0)
    nblk = _HALF // _BT

    @pl.when(i == 0)
    def _prologue():
        for s in range(_K):
            _copy_in(x_hbm, xbuf, sems, s, s)

    slot = jax.lax.rem(i, _K)

    def head(xblk):
        logits = jax.lax.dot_general(
            xblk, w_ref[...],
            dimension_numbers=(((1,), (1,)), ((), ())),
            preferred_element_type=jnp.float32,
        ) + b_ref[...]
        m = jnp.max(logits, axis=-1, keepdims=True)
        e = jnp.exp(logits - m)
        s = jnp.sum(e, axis=-1, keepdims=True)
        return e / s, jnp.sum(logits * logits)

    pa, za = head(xa_ref[...])
    probs_ref[0] = pa

    pltpu.make_async_copy(
        x_hbm.at[pl.ds(_HALF + i * _BT, _BT), :],
        xbuf.at[slot],
        sems.at[slot],
    ).wait()
    pb, zb = head(xbuf[slot])
    probs_ref[1] = pb

    nxt = i + _K

    @pl.when(nxt < nblk)
    def _refill():
        _copy_in(x_hbm, xbuf, sems, nxt, slot)

    part = (za + zb).reshape(1, 1)

    @pl.when(i == 0)
    def _init():
        zacc_ref[...] = part

    @pl.when(i > 0)
    def _acc():
        zacc_ref[...] += part

    @pl.when(i == nblk - 1)
    def _fin():
        z_ref[...] = zacc_ref[...] * (_COEFF / (_TOK * _NE))


def kernel(x, W, b):
    nblk = _HALF // _BT
    probs2, z = pl.pallas_call(
        _router_kernel,
        grid=(nblk,),
        in_specs=[
            pl.BlockSpec((_BT, _EMB), lambda i: (i, 0)),
            pl.BlockSpec(memory_space=pltpu.MemorySpace.HBM),
            pl.BlockSpec((_NE, _EMB), lambda i: (0, 0)),
            pl.BlockSpec((1, _NE), lambda i: (0, 0)),
        ],
        out_specs=[
            pl.BlockSpec((2, _BT, _NE), lambda i: (0, i, 0)),
            pl.BlockSpec((1, 1), lambda i: (0, 0)),
        ],
        out_shape=[
            jax.ShapeDtypeStruct((2, _HALF, _NE), jnp.float32),
            jax.ShapeDtypeStruct((1, 1), jnp.float32),
        ],
        scratch_shapes=[
            pltpu.VMEM((_K, _BT, _EMB), jnp.float32),
            pltpu.SemaphoreType.DMA((_K,)),
            pltpu.VMEM((1, 1), jnp.float32),
        ],
        compiler_params=pltpu.CompilerParams(
            dimension_semantics=("arbitrary",),
        ),
    )(x, x, W, b.reshape(1, _NE))
    return (probs2.reshape(_TOK, _NE), z.reshape(()))


# speedup vs baseline: 1.1790x; 1.0759x over previous
import jax
import jax.numpy as jnp
from jax.experimental import pallas as pl
from jax.experimental.pallas import tpu as pltpu

_EMB = 2048
_NE = 64
_TOK = 16384
_COEFF = 0.001
_BT = 1024


def _router_kernel(x_ref, w_ref, b_ref, probs_ref, z_ref, zacc_ref):
    i = pl.program_id(0)
    nblk = _TOK // _BT
    logits = jax.lax.dot_general(
        x_ref[...], w_ref[...],
        dimension_numbers=(((1,), (1,)), ((), ())),
        preferred_element_type=jnp.float32,
    ) + b_ref[...]
    m = jnp.max(logits, axis=-1, keepdims=True)
    e = jnp.exp(logits - m)
    s = jnp.sum(e, axis=-1, keepdims=True)
    probs_ref[...] = e / s
    part = jnp.sum(logits * logits).reshape(1, 1)

    @pl.when(i == 0)
    def _init():
        zacc_ref[...] = part

    @pl.when(i > 0)
    def _acc():
        zacc_ref[...] += part

    @pl.when(i == nblk - 1)
    def _fin():
        z_ref[...] = zacc_ref[...] * (_COEFF / (_TOK * _NE))


def kernel(x, W, b):
    nblk = _TOK // _BT
    probs, z = pl.pallas_call(
        _router_kernel,
        grid=(nblk,),
        in_specs=[
            pl.BlockSpec((_BT, _EMB), lambda i: (i, 0)),
            pl.BlockSpec((_NE, _EMB), lambda i: (0, 0)),
            pl.BlockSpec((1, _NE), lambda i: (0, 0)),
        ],
        out_specs=[
            pl.BlockSpec((_BT, _NE), lambda i: (i, 0)),
            pl.BlockSpec((1, 1), lambda i: (0, 0)),
        ],
        out_shape=[
            jax.ShapeDtypeStruct((_TOK, _NE), jnp.float32),
            jax.ShapeDtypeStruct((1, 1), jnp.float32),
        ],
        scratch_shapes=[
            pltpu.VMEM((1, 1), jnp.float32),
        ],
        compiler_params=pltpu.CompilerParams(
            dimension_semantics=("arbitrary",),
        ),
    )(x, W, b.reshape(1, _NE))
    return (probs, z.reshape(()))
